# Initial kernel scaffold; baseline (speedup 1.0000x reference)
import jax
import jax.numpy as jnp
from jax.experimental import pallas as pl


def kernel(x, node_type_ids, edge_index, type_emb, W_l, b_l, W_r):
    raise NotImplementedError("write your pallas kernel here")


# speedup vs baseline: 6.0619x; 6.0619x over previous
import functools

import jax
import jax.numpy as jnp
from jax import lax
from jax.experimental import pallas as pl
from jax.experimental.pallas import tpu as pltpu
from jax.experimental.pallas import tpu_sc as plsc

N_NODES = 10000
N_EDGES = 320000
D_FEAT = 128
NUM_NODE_TYPES = 8
OUT_CH = 128

NP_ = 10240
NC = 2
NS = 16
NW = NC * NS
EPT = N_EDGES // NS
CH = 96
NFULL = EPT // CH
REM = EPT - NFULL * CH
RPT = NP_ // NS
RB = 1280
GRID = NP_ // RB


def _proj_body(x_ref, ids_ref, wl_ref, wr_ref, tl_ref, tr_ref, p_ref, r_ref):
    x = x_ref[...]
    oh = (ids_ref[...] == lax.broadcasted_iota(jnp.int32, (RB, NUM_NODE_TYPES), 1)
          ).astype(jnp.float32)
    p_ref[...] = (jnp.dot(x, wl_ref[...], preferred_element_type=jnp.float32)
                  + jnp.dot(oh, tl_ref[...], preferred_element_type=jnp.float32))
    r_ref[...] = (jnp.dot(x, wr_ref[...], preferred_element_type=jnp.float32)
                  + jnp.dot(oh, tr_ref[...], preferred_element_type=jnp.float32))


_proj = pl.pallas_call(
    _proj_body,
    grid=(GRID,),
    in_specs=[
        pl.BlockSpec((RB, D_FEAT), lambda i: (i, 0)),
        pl.BlockSpec((RB, 1), lambda i: (i, 0)),
        pl.BlockSpec((D_FEAT, OUT_CH), lambda i: (0, 0)),
        pl.BlockSpec((D_FEAT, OUT_CH), lambda i: (0, 0)),
        pl.BlockSpec((NUM_NODE_TYPES, OUT_CH), lambda i: (0, 0)),
        pl.BlockSpec((NUM_NODE_TYPES, OUT_CH), lambda i: (0, 0)),
    ],
    out_specs=[pl.BlockSpec((RB, OUT_CH), lambda i: (i, 0))] * 2,
    out_shape=[jax.ShapeDtypeStruct((NP_, OUT_CH), jnp.float32)] * 2,
)


def _sc_body(p_hbm, src_hbm, dst_hbm, agg_hbm,
             acc, rows0, rows1, s0, s1, d0, d1, sr, dr, sem0, sem1):
    c = lax.axis_index("c")
    s = lax.axis_index("s")
    ebase = s * EPT
    zero16 = jnp.zeros((16,), jnp.float32)
    ones16 = jnp.ones((16,), jnp.float32)

    def _zb(i, carry):
        for j in range(OUT_CH // 16):
            rows0[i, pl.ds(j * 16, 16)] = zero16
        return carry
    lax.fori_loop(0, CH, _zb, None)
    nz = RPT // CH
    for j in range(nz):
        pltpu.sync_copy(rows0, acc.at[pl.ds(s * RPT + j * CH, CH)])
    zrem = RPT - nz * CH
    if zrem:
        pltpu.sync_copy(rows0.at[pl.ds(0, zrem)],
                        acc.at[pl.ds(s * RPT + nz * CH, zrem)])

    @pl.when(c == 1)
    def _():
        def _ob(i, carry):
            for j in range(OUT_CH // 16):
                rows0[i, pl.ds(j * 16, 16)] = ones16
            return carry
        lax.fori_loop(0, CH, _ob, None)

    plsc.subcore_barrier()

    @pl.when(c == 0)
    def _():
        def _pair(t, carry):
            off0 = ebase + 2 * t * CH
            off1 = off0 + CH
            pltpu.sync_copy(src_hbm.at[pl.ds(off0, CH)], s0)
            pltpu.sync_copy(dst_hbm.at[pl.ds(off0, CH)], d0)
            h0 = pltpu.async_copy(p_hbm.at[s0], rows0, sem0)
            pltpu.sync_copy(src_hbm.at[pl.ds(off1, CH)], s1)
            pltpu.sync_copy(dst_hbm.at[pl.ds(off1, CH)], d1)
            h1 = pltpu.async_copy(p_hbm.at[s1], rows1, sem1)
            h0.wait()
            pltpu.sync_copy(rows0, acc.at[d0], add=True)
            h1.wait()
            pltpu.sync_copy(rows1, acc.at[d1], add=True)
            return carry
        lax.fori_loop(0, NFULL // 2, _pair, None)
        roff = ebase + NFULL * CH
        pltpu.sync_copy(src_hbm.at[pl.ds(roff, REM)], sr)
        pltpu.sync_copy(dst_hbm.at[pl.ds(roff, REM)], dr)
        pltpu.async_copy(p_hbm.at[sr], rows1.at[pl.ds(0, REM)], sem0).wait()
        pltpu.sync_copy(rows1.at[pl.ds(0, REM)], acc.at[dr], add=True)

    @pl.when(c == 1)
    def _():
        def _one(g, carry):
            off = ebase + g * CH
            pltpu.sync_copy(dst_hbm.at[pl.ds(off, CH)], d0)
            pltpu.sync_copy(rows0, acc.at[d0], add=True)
            return carry
        lax.fori_loop(0, NFULL, _one, None)
        roff = ebase + NFULL * CH
        pltpu.sync_copy(dst_hbm.at[pl.ds(roff, REM)], dr)
        pltpu.sync_copy(rows0.at[pl.ds(0, REM)], acc.at[dr], add=True)

    plsc.subcore_barrier()

    pltpu.sync_copy(acc.at[pl.ds(s * RPT, RPT)],
                    agg_hbm.at[pl.ds(c * NP_ + s * RPT, RPT)])


_sc_agg = functools.partial(
    pl.kernel,
    mesh=plsc.VectorSubcoreMesh(core_axis_name="c", subcore_axis_name="s"),
    out_type=jax.ShapeDtypeStruct((NC * NP_, OUT_CH), jnp.float32),
    scratch_types=[
        pltpu.VMEM_SHARED((NP_, OUT_CH), jnp.float32),
        pltpu.VMEM((CH, OUT_CH), jnp.float32),
        pltpu.VMEM((CH, OUT_CH), jnp.float32),
        pltpu.VMEM((CH,), jnp.int32),
        pltpu.VMEM((CH,), jnp.int32),
        pltpu.VMEM((CH,), jnp.int32),
        pltpu.VMEM((CH,), jnp.int32),
        pltpu.VMEM((REM,), jnp.int32),
        pltpu.VMEM((REM,), jnp.int32),
        pltpu.SemaphoreType.DMA,
        pltpu.SemaphoreType.DMA,
    ],
)(_sc_body)


def _final_body(agg_ref, r_ref, o_ref):
    a = agg_ref[0]
    cnt = agg_ref[1][:, 0:1]
    o_ref[...] = jnp.maximum(a / jnp.maximum(cnt, 1.0) + r_ref[...], 0.0)


_final = pl.pallas_call(
    _final_body,
    grid=(GRID,),
    in_specs=[
        pl.BlockSpec((NC, RB, OUT_CH), lambda i: (0, i, 0)),
        pl.BlockSpec((RB, OUT_CH), lambda i: (i, 0)),
    ],
    out_specs=pl.BlockSpec((RB, OUT_CH), lambda i: (i, 0)),
    out_shape=jax.ShapeDtypeStruct((NP_, OUT_CH), jnp.float32),
)


def kernel(x, node_type_ids, edge_index, type_emb, W_l, b_l, W_r):
    ids = node_type_ids.astype(jnp.int32)
    src = edge_index[0].astype(jnp.int32)
    dst = edge_index[1].astype(jnp.int32)
    wl_x = W_l[:, :D_FEAT].T
    wr_x = W_r[:, :D_FEAT].T
    tl = type_emb @ W_l[:, D_FEAT:].T
    tr = type_emb @ W_r[:, D_FEAT:].T + b_l[None, :]
    xp = jnp.pad(x, ((0, NP_ - N_NODES), (0, 0)))
    idsp = jnp.pad(ids, (0, NP_ - N_NODES))[:, None]
    p, r = _proj(xp, idsp, wl_x, wr_x, tl, tr)
    agg = _sc_agg(p, src, dst)
    out = _final(agg.reshape(NC, NP_, OUT_CH), r)
    return out[:N_NODES]


# speedup vs baseline: 7.8896x; 1.3015x over previous
import functools

import jax
import jax.numpy as jnp
from jax import lax
from jax.experimental import pallas as pl
from jax.experimental.pallas import tpu as pltpu
from jax.experimental.pallas import tpu_sc as plsc

N_NODES = 10000
N_EDGES = 320000
D_FEAT = 128
NUM_NODE_TYPES = 8
OUT_CH = 128

NP_ = 10240
NC = 2
NS = 16
NW = NC * NS
EPT = N_EDGES // NS
CH = 128
NFULL = EPT // CH
REM = EPT - NFULL * CH
RPT = NP_ // NS
RB = 1280
GRID = NP_ // RB


def _proj_body(x_ref, ids_ref, wl_ref, wr_ref, tl_ref, tr_ref, p_ref, r_ref):
    x = x_ref[...]
    oh = (ids_ref[...] == lax.broadcasted_iota(jnp.int32, (RB, NUM_NODE_TYPES), 1)
          ).astype(jnp.float32)
    p_ref[...] = (jnp.dot(x, wl_ref[...], preferred_element_type=jnp.float32)
                  + jnp.dot(oh, tl_ref[...], preferred_element_type=jnp.float32))
    r_ref[...] = (jnp.dot(x, wr_ref[...], preferred_element_type=jnp.float32)
                  + jnp.dot(oh, tr_ref[...], preferred_element_type=jnp.float32))


_proj = pl.pallas_call(
    _proj_body,
    grid=(GRID,),
    in_specs=[
        pl.BlockSpec((RB, D_FEAT), lambda i: (i, 0)),
        pl.BlockSpec((RB, 1), lambda i: (i, 0)),
        pl.BlockSpec((D_FEAT, OUT_CH), lambda i: (0, 0)),
        pl.BlockSpec((D_FEAT, OUT_CH), lambda i: (0, 0)),
        pl.BlockSpec((NUM_NODE_TYPES, OUT_CH), lambda i: (0, 0)),
        pl.BlockSpec((NUM_NODE_TYPES, OUT_CH), lambda i: (0, 0)),
    ],
    out_specs=[pl.BlockSpec((RB, OUT_CH), lambda i: (i, 0))] * 2,
    out_shape=[jax.ShapeDtypeStruct((NP_, OUT_CH), jnp.float32)] * 2,
)


def _sc_body(p_hbm, src_hbm, dst_hbm, agg_hbm,
             acc, rows0, rows1, s0, s1, d0, d1, sr, dr, sem0, sem1):
    c = lax.axis_index("c")
    s = lax.axis_index("s")
    ebase = s * EPT
    zero16 = jnp.zeros((16,), jnp.float32)
    ones16 = jnp.ones((16,), jnp.float32)

    def _zb(i, carry):
        for j in range(OUT_CH // 16):
            rows0[i, pl.ds(j * 16, 16)] = zero16
        return carry
    lax.fori_loop(0, CH, _zb, None)
    nz = RPT // CH
    for j in range(nz):
        pltpu.sync_copy(rows0, acc.at[pl.ds(s * RPT + j * CH, CH)])
    zrem = RPT - nz * CH
    if zrem:
        pltpu.sync_copy(rows0.at[pl.ds(0, zrem)],
                        acc.at[pl.ds(s * RPT + nz * CH, zrem)])

    @pl.when(c == 1)
    def _():
        def _ob(i, carry):
            for j in range(OUT_CH // 16):
                rows0[i, pl.ds(j * 16, 16)] = ones16
            return carry
        lax.fori_loop(0, CH, _ob, None)

    plsc.subcore_barrier()

    @pl.when(c == 0)
    def _():
        pltpu.sync_copy(src_hbm.at[pl.ds(ebase, CH)], s0)
        pltpu.sync_copy(dst_hbm.at[pl.ds(ebase, CH)], d0)
        pltpu.async_copy(p_hbm.at[s0], rows0, sem0)

        def _pair(t, carry):
            off1 = ebase + (2 * t + 1) * CH
            off2 = off1 + CH
            pltpu.sync_copy(src_hbm.at[pl.ds(off1, CH)], s1)
            pltpu.sync_copy(dst_hbm.at[pl.ds(off1, CH)], d1)
            pltpu.async_copy(p_hbm.at[s1], rows1, sem1)
            pltpu.make_async_copy(p_hbm.at[s0], rows0, sem0).wait()
            pltpu.sync_copy(rows0, acc.at[d0], add=True)
            pltpu.sync_copy(src_hbm.at[pl.ds(off2, CH)], s0)
            pltpu.sync_copy(dst_hbm.at[pl.ds(off2, CH)], d0)
            pltpu.async_copy(p_hbm.at[s0], rows0, sem0)
            pltpu.make_async_copy(p_hbm.at[s1], rows1, sem1).wait()
            pltpu.sync_copy(rows1, acc.at[d1], add=True)
            return carry
        lax.fori_loop(0, NFULL // 2, _pair, None)

        pltpu.make_async_copy(p_hbm.at[s0], rows0, sem0).wait()
        roff = ebase + NFULL * CH
        pltpu.sync_copy(src_hbm.at[pl.ds(roff, REM)], sr)
        pltpu.sync_copy(dst_hbm.at[pl.ds(roff, REM)], dr)
        pltpu.async_copy(p_hbm.at[sr], rows1.at[pl.ds(0, REM)], sem0).wait()
        pltpu.sync_copy(rows1.at[pl.ds(0, REM)], acc.at[dr], add=True)

    @pl.when(c == 1)
    def _():
        def _one(g, carry):
            off = ebase + g * CH
            pltpu.sync_copy(dst_hbm.at[pl.ds(off, CH)], d0)
            pltpu.sync_copy(rows0, acc.at[d0], add=True)
            return carry
        lax.fori_loop(0, NFULL, _one, None)
        roff = ebase + NFULL * CH
        pltpu.sync_copy(dst_hbm.at[pl.ds(roff, REM)], dr)
        pltpu.sync_copy(rows0.at[pl.ds(0, REM)], acc.at[dr], add=True)

    plsc.subcore_barrier()

    pltpu.sync_copy(acc.at[pl.ds(s * RPT, RPT)],
                    agg_hbm.at[pl.ds(c * NP_ + s * RPT, RPT)])


_sc_agg = functools.partial(
    pl.kernel,
    mesh=plsc.VectorSubcoreMesh(core_axis_name="c", subcore_axis_name="s"),
    out_type=jax.ShapeDtypeStruct((NC * NP_, OUT_CH), jnp.float32),
    scratch_types=[
        pltpu.VMEM_SHARED((NP_, OUT_CH), jnp.float32),
        pltpu.VMEM((CH, OUT_CH), jnp.float32),
        pltpu.VMEM((CH, OUT_CH), jnp.float32),
        pltpu.VMEM((CH,), jnp.int32),
        pltpu.VMEM((CH,), jnp.int32),
        pltpu.VMEM((CH,), jnp.int32),
        pltpu.VMEM((CH,), jnp.int32),
        pltpu.VMEM((REM,), jnp.int32),
        pltpu.VMEM((REM,), jnp.int32),
        pltpu.SemaphoreType.DMA,
        pltpu.SemaphoreType.DMA,
    ],
)(_sc_body)


def _final_body(agg_ref, r_ref, o_ref):
    a = agg_ref[0]
    cnt = agg_ref[1][:, 0:1]
    o_ref[...] = jnp.maximum(a / jnp.maximum(cnt, 1.0) + r_ref[...], 0.0)


_final = pl.pallas_call(
    _final_body,
    grid=(GRID,),
    in_specs=[
        pl.BlockSpec((NC, RB, OUT_CH), lambda i: (0, i, 0)),
        pl.BlockSpec((RB, OUT_CH), lambda i: (i, 0)),
    ],
    out_specs=pl.BlockSpec((RB, OUT_CH), lambda i: (i, 0)),
    out_shape=jax.ShapeDtypeStruct((NP_, OUT_CH), jnp.float32),
)


def kernel(x, node_type_ids, edge_index, type_emb, W_l, b_l, W_r):
    ids = node_type_ids.astype(jnp.int32)
    src = jnp.pad(edge_index[0].astype(jnp.int32), (0, CH))
    dst = jnp.pad(edge_index[1].astype(jnp.int32), (0, CH))
    wl_x = W_l[:, :D_FEAT].T
    wr_x = W_r[:, :D_FEAT].T
    tl = type_emb @ W_l[:, D_FEAT:].T
    tr = type_emb @ W_r[:, D_FEAT:].T + b_l[None, :]
    xp = jnp.pad(x, ((0, NP_ - N_NODES), (0, 0)))
    idsp = jnp.pad(ids, (0, NP_ - N_NODES))[:, None]
    p, r = _proj(xp, idsp, wl_x, wr_x, tl, tr)
    agg = _sc_agg(p, src, dst)
    out = _final(agg.reshape(NC, NP_, OUT_CH), r)
    return out[:N_NODES]
